# baseline (speedup 1.0000x reference)
import functools

import jax
import jax.numpy as jnp
from jax import lax
from jax.experimental import pallas as pl
from jax.experimental.pallas import tpu as pltpu
from jax.experimental.pallas import tpu_sc as plsc

_N = 10000
_P = 2
_E = 320000
_F = 128
_SH = 64
_KC = 32
_NT = 16
_C = 100
_NCH = _E // (_NT * _C)
_BC = 20
_NBLK = _NCH // _BC
_RPT = _N // _NT


def _pre_matmul(x1, x2, W):
    def body(x1_ref, x2_ref, w_ref, o1_ref, o2_ref):
        w = w_ref[...]
        o1_ref[...] = jnp.dot(x1_ref[...], w, preferred_element_type=jnp.float32)
        o2_ref[...] = jnp.dot(x2_ref[...], w, preferred_element_type=jnp.float32)

    return pl.pallas_call(
        body,
        out_shape=(jax.ShapeDtypeStruct((_N, _F), jnp.float32),
                   jax.ShapeDtypeStruct((_N, _F), jnp.float32)),
    )(x1, x2, W)


def _sc_pass(xw, adjs_r, ones16, with_deg):
    mesh = plsc.VectorSubcoreMesh(core_axis_name="c", subcore_axis_name="s",
                                  num_cores=2, num_subcores=_NT)

    out_type = [jax.ShapeDtypeStruct((_P, _N, _F), jnp.float32)]
    if with_deg:
        out_type.append(jax.ShapeDtypeStruct((_P, _N, 16), jnp.float32))

    @functools.partial(
        pl.kernel,
        out_type=tuple(out_type),
        mesh=mesh,
        scratch_types=[
            pltpu.VMEM_SHARED((_N, _F), jnp.float32),
            pltpu.VMEM_SHARED((_N, 16), jnp.float32),
            pltpu.VMEM((_BC, _C), jnp.int32),
            pltpu.VMEM((_BC, _C), jnp.int32),
            pltpu.VMEM((_BC, _C), jnp.int32),
            pltpu.VMEM((_BC, _C), jnp.int32),
            pltpu.VMEM((_C, _F), jnp.float32),
            pltpu.VMEM((_C, _F), jnp.float32),
            pltpu.VMEM((_C, 16), jnp.float32),
            pltpu.SemaphoreType.DMA,
            pltpu.SemaphoreType.DMA,
            pltpu.SemaphoreType.DMA,
            pltpu.SemaphoreType.DMA,
            pltpu.SemaphoreType.DMA,
        ],
        compiler_params=pltpu.CompilerParams(use_tc_tiling_on_sc=False),
    )
    def k(xw_hbm, adj_hbm, ones_hbm, out_hbm, *rest):
        if with_deg:
            deg_hbm = rest[0]
            rest = rest[1:]
        (acc, dacc, src0_v, dst0_v, src1_v, dst1_v, rows0_v, rows1_v, ones_v,
         gsem0, gsem1, isem0, isem1, dsem) = rest
        rel = lax.axis_index("c")
        tid = lax.axis_index("s")
        rows = pl.ds(tid * _RPT, _RPT)

        def stage_idx(b, src_v, dst_v, isem):
            pltpu.async_copy(adj_hbm.at[rel, 0, tid, pl.ds(b * _BC, _BC)],
                             src_v, isem)
            pltpu.async_copy(adj_hbm.at[rel, 1, tid, pl.ds(b * _BC, _BC)],
                             dst_v, isem)

        def wait_idx(b, src_v, dst_v, isem):
            pltpu.make_async_copy(adj_hbm.at[rel, 0, tid, pl.ds(b * _BC, _BC)],
                                  src_v, isem).wait()
            pltpu.make_async_copy(adj_hbm.at[rel, 1, tid, pl.ds(b * _BC, _BC)],
                                  dst_v, isem).wait()

        def run_block(src_v, dst_v, with_deg):
            pltpu.async_copy(xw_hbm.at[src_v.at[0]], rows0_v, gsem0)

            @pl.loop(0, _BC, step=2)
            def _(j):
                pltpu.async_copy(xw_hbm.at[src_v.at[j + 1]], rows1_v, gsem1)
                pltpu.make_async_copy(xw_hbm.at[src_v.at[j]], rows0_v,
                                      gsem0).wait()
                pltpu.sync_copy(rows0_v, acc.at[dst_v.at[j]], add=True)
                if with_deg:
                    pltpu.async_copy(ones_v, dacc.at[dst_v.at[j]], dsem,
                                     add=True)

                @pl.when(j + 2 < _BC)
                def _():
                    pltpu.async_copy(xw_hbm.at[src_v.at[j + 2]], rows0_v,
                                     gsem0)

                pltpu.make_async_copy(xw_hbm.at[src_v.at[j + 1]], rows1_v,
                                      gsem1).wait()
                pltpu.sync_copy(rows1_v, acc.at[dst_v.at[j + 1]], add=True)
                if with_deg:
                    pltpu.async_copy(ones_v, dacc.at[dst_v.at[j + 1]],
                                     dsem, add=True)

        def edge_pass(xw_hbm, with_deg):
            stage_idx(0, src0_v, dst0_v, isem0)

            @pl.loop(0, _NBLK, step=2)
            def _(b):
                stage_idx(b + 1, src1_v, dst1_v, isem1)
                wait_idx(b, src0_v, dst0_v, isem0)
                run_block(src0_v, dst0_v, with_deg)

                @pl.when(b + 2 < _NBLK)
                def _():
                    stage_idx(b + 2, src0_v, dst0_v, isem0)

                wait_idx(b + 1, src1_v, dst1_v, isem1)
                run_block(src1_v, dst1_v, with_deg)

            if with_deg:
                @pl.loop(0, _NBLK * _BC)
                def _(i):
                    pltpu.make_async_copy(ones_v, dacc.at[dst1_v.at[0]],
                                          dsem).wait()

        if with_deg:
            pltpu.sync_copy(ones_hbm.at[pl.ds(0, _C)], ones_v)
            pltpu.sync_copy(ones_hbm.at[rows], dacc.at[rows])
        pltpu.sync_copy(xw_hbm.at[rows], acc.at[rows])
        plsc.subcore_barrier()
        edge_pass(xw_hbm, with_deg=with_deg)
        plsc.subcore_barrier()
        pltpu.sync_copy(acc.at[rows], out_hbm.at[rel, rows])
        if with_deg:
            pltpu.sync_copy(dacc.at[rows], deg_hbm.at[rel, rows])

    out = k(xw, adjs_r, ones16)
    return out if with_deg else out[0]


def _att(sums, degP, Wsem, q2):
    def body(s_ref, d_ref, wsem_ref, q_ref, h_ref, sm_ref):
        dinv = 1.0 / d_ref[0]
        h = jnp.maximum(s_ref[0] * dinv, 0.0)
        h_ref[0] = h
        t = jnp.tanh(jnp.dot(h, wsem_ref[...], preferred_element_type=jnp.float32))
        att = jnp.dot(t, q_ref[...], preferred_element_type=jnp.float32)
        sm_ref[0] = jnp.mean(att, axis=0, keepdims=True)

    return pl.pallas_call(
        body,
        grid=(_P,),
        in_specs=[
            pl.BlockSpec((1, _N, _F), lambda p: (p, 0, 0)),
            pl.BlockSpec((1, _N, 1), lambda p: (p, 0, 0)),
            pl.BlockSpec((_F, _SH), lambda p: (0, 0)),
            pl.BlockSpec((_SH, 1), lambda p: (0, 0)),
        ],
        out_specs=[
            pl.BlockSpec((1, _N, _F), lambda p: (p, 0, 0)),
            pl.BlockSpec((1, 1, 1), lambda p: (p, 0, 0)),
        ],
        out_shape=(jax.ShapeDtypeStruct((_P, _N, _F), jnp.float32),
                   jax.ShapeDtypeStruct((_P, 1, 1), jnp.float32)),
    )(sums, degP, Wsem, q2)


def _softmax_combine(h_ref, sm_ref):
    s0 = sm_ref[0]
    s1 = sm_ref[1]
    m = jnp.maximum(s0, s1)
    e0 = jnp.exp(s0 - m)
    e1 = jnp.exp(s1 - m)
    return (e0 * h_ref[0] + e1 * h_ref[1]) / (e0 + e1)


def _head1(hst, sm, init0, ct):
    def body(h_ref, sm_ref, mu0_ref, ct_ref,
             h1_ref, c_ref, mu_ref, r_ref, dist_ref):
        h1 = _softmax_combine(h_ref, sm_ref)
        h1_ref[...] = h1
        c_ref[...] = jax.nn.sigmoid(jnp.mean(h1, axis=0, keepdims=True))
        ct = ct_ref[...]
        nrm = lax.rsqrt(jnp.sum(h1 * h1, axis=1, keepdims=True))
        data = h1 * nrm
        ones_col = jnp.ones((_N, 1), jnp.float32)
        mu = mu0_ref[...]
        for _ in range(2):
            dist = lax.dot_general(data, mu, (((1,), (1,)), ((), ())),
                                   preferred_element_type=jnp.float32)
            z = ct * dist
            ex = jnp.exp(z - jnp.max(z, axis=1, keepdims=True))
            r = ex / jnp.sum(ex, axis=1, keepdims=True)
            cm = lax.dot_general(r, data, (((0,), (0,)), ((), ())),
                                 preferred_element_type=jnp.float32)
            cr = lax.dot_general(r, ones_col, (((0,), (0,)), ((), ())),
                                 preferred_element_type=jnp.float32)
            mu = cm / cr
        dist = lax.dot_general(data, mu, (((1,), (1,)), ((), ())),
                               preferred_element_type=jnp.float32)
        z = ct * dist
        ex = jnp.exp(z - jnp.max(z, axis=1, keepdims=True))
        r = ex / jnp.sum(ex, axis=1, keepdims=True)
        mu_ref[...] = mu
        r_ref[...] = r
        dist_ref[...] = dist

    return pl.pallas_call(
        body,
        out_shape=(jax.ShapeDtypeStruct((_N, _F), jnp.float32),
                   jax.ShapeDtypeStruct((1, _F), jnp.float32),
                   jax.ShapeDtypeStruct((_KC, _F), jnp.float32),
                   jax.ShapeDtypeStruct((_N, _KC), jnp.float32),
                   jax.ShapeDtypeStruct((_N, _KC), jnp.float32)),
        compiler_params=pltpu.CompilerParams(vmem_limit_bytes=63 << 20),
    )(hst, sm, init0, ct)


def _head2(hst, sm):
    def body(h_ref, sm_ref, h2_ref):
        h2_ref[...] = _softmax_combine(h_ref, sm_ref)

    return pl.pallas_call(
        body,
        out_shape=jax.ShapeDtypeStruct((_N, _F), jnp.float32),
    )(hst, sm)


def kernel(seq1, seq2, adjs, sparse, msk, samp_bias1, samp_bias2, K,
           cluster_temp, W, Wsem, q):
    x1 = seq1[0]
    x2 = seq2[0]
    xw1, xw2 = _pre_matmul(x1, x2, W)
    adjs_r = adjs.reshape(_P, 2, _NT, _NCH, _C)
    ones16 = jnp.ones((_N, 16), jnp.float32)
    sums1, degs = _sc_pass(xw1, adjs_r, ones16, with_deg=True)
    sums2 = _sc_pass(xw2, adjs_r, ones16, with_deg=False)
    degP = degs[:, :, 0:1]
    ct = jnp.asarray(cluster_temp, jnp.float32).reshape(1, 1)
    init0 = jax.random.uniform(jax.random.key(42), (_KC, _F), dtype=jnp.float32)
    q2 = q.reshape(_SH, 1)
    hst1, sm1 = _att(sums1, degP, Wsem, q2)
    h1, c, mu, r, dist = _head1(hst1, sm1, init0, ct)
    hst2, sm2 = _att(sums2, degP, Wsem, q2)
    h2 = _head2(hst2, sm2)
    return (h1[None], h2[None], c, mu, r, dist)


# speedup vs baseline: 1.0446x; 1.0446x over previous
import functools

import jax
import jax.numpy as jnp
from jax import lax
from jax.experimental import pallas as pl
from jax.experimental.pallas import tpu as pltpu
from jax.experimental.pallas import tpu_sc as plsc

_N = 10000
_P = 2
_E = 320000
_F = 128
_SH = 64
_KC = 32
_NT = 16
_C = 125
_NCH = _E // (_NT * _C)
_BC = 10
_NBLK = _NCH // _BC
_RPT = _N // _NT


def _pre_matmul(x1, x2, W):
    def body(x1_ref, x2_ref, w_ref, o1_ref, o2_ref):
        w = w_ref[...]
        o1_ref[...] = jnp.dot(x1_ref[...], w, preferred_element_type=jnp.float32)
        o2_ref[...] = jnp.dot(x2_ref[...], w, preferred_element_type=jnp.float32)

    return pl.pallas_call(
        body,
        out_shape=(jax.ShapeDtypeStruct((_N, _F), jnp.float32),
                   jax.ShapeDtypeStruct((_N, _F), jnp.float32)),
    )(x1, x2, W)


def _sc_msgpass(xw1, xw2, adjs_r, ones16):
    mesh = plsc.VectorSubcoreMesh(core_axis_name="c", subcore_axis_name="s",
                                  num_cores=2, num_subcores=_NT)

    @functools.partial(
        pl.kernel,
        out_type=(jax.ShapeDtypeStruct((2, _P, _N, _F), jnp.float32),
                  jax.ShapeDtypeStruct((_P, _N, 16), jnp.float32)),
        mesh=mesh,
        scratch_types=[
            pltpu.VMEM_SHARED((_N, _F), jnp.float32),
            pltpu.VMEM_SHARED((_N, 16), jnp.float32),
            pltpu.VMEM((_BC, _C), jnp.int32),
            pltpu.VMEM((_BC, _C), jnp.int32),
            pltpu.VMEM((_BC, _C), jnp.int32),
            pltpu.VMEM((_BC, _C), jnp.int32),
            pltpu.VMEM((_C, _F), jnp.float32),
            pltpu.VMEM((_C, _F), jnp.float32),
            pltpu.VMEM((_C, 16), jnp.float32),
            pltpu.SemaphoreType.DMA,
            pltpu.SemaphoreType.DMA,
            pltpu.SemaphoreType.DMA,
            pltpu.SemaphoreType.DMA,
            pltpu.SemaphoreType.DMA,
        ],
        compiler_params=pltpu.CompilerParams(use_tc_tiling_on_sc=False),
    )
    def k(xw1_hbm, xw2_hbm, adj_hbm, ones_hbm, out_hbm, deg_hbm,
          acc, dacc, src0_v, dst0_v, src1_v, dst1_v, rows0_v, rows1_v, ones_v,
          gsem0, gsem1, isem0, isem1, dsem):
        rel = lax.axis_index("c")
        tid = lax.axis_index("s")
        rows = pl.ds(tid * _RPT, _RPT)

        def stage_idx(b, src_v, dst_v, isem):
            pltpu.async_copy(adj_hbm.at[rel, 0, tid, pl.ds(b * _BC, _BC)],
                             src_v, isem)
            pltpu.async_copy(adj_hbm.at[rel, 1, tid, pl.ds(b * _BC, _BC)],
                             dst_v, isem)

        def wait_idx(b, src_v, dst_v, isem):
            pltpu.make_async_copy(adj_hbm.at[rel, 0, tid, pl.ds(b * _BC, _BC)],
                                  src_v, isem).wait()
            pltpu.make_async_copy(adj_hbm.at[rel, 1, tid, pl.ds(b * _BC, _BC)],
                                  dst_v, isem).wait()

        def run_block(xw_hbm, src_v, dst_v, with_deg):
            pltpu.async_copy(xw_hbm.at[src_v.at[0]], rows0_v, gsem0)

            @pl.loop(0, _BC, step=2)
            def _(j):
                pltpu.async_copy(xw_hbm.at[src_v.at[j + 1]], rows1_v, gsem1)
                pltpu.make_async_copy(xw_hbm.at[src_v.at[j]], rows0_v,
                                      gsem0).wait()
                pltpu.sync_copy(rows0_v, acc.at[dst_v.at[j]], add=True)
                if with_deg:
                    pltpu.async_copy(ones_v, dacc.at[dst_v.at[j]], dsem,
                                     add=True)

                @pl.when(j + 2 < _BC)
                def _():
                    pltpu.async_copy(xw_hbm.at[src_v.at[j + 2]], rows0_v,
                                     gsem0)

                pltpu.make_async_copy(xw_hbm.at[src_v.at[j + 1]], rows1_v,
                                      gsem1).wait()
                pltpu.sync_copy(rows1_v, acc.at[dst_v.at[j + 1]], add=True)
                if with_deg:
                    pltpu.async_copy(ones_v, dacc.at[dst_v.at[j + 1]],
                                     dsem, add=True)

        def edge_pass(xw_hbm, with_deg):
            stage_idx(0, src0_v, dst0_v, isem0)

            @pl.loop(0, _NBLK, step=2)
            def _(b):
                stage_idx(b + 1, src1_v, dst1_v, isem1)
                wait_idx(b, src0_v, dst0_v, isem0)
                run_block(xw_hbm, src0_v, dst0_v, with_deg)

                @pl.when(b + 2 < _NBLK)
                def _():
                    stage_idx(b + 2, src0_v, dst0_v, isem0)

                wait_idx(b + 1, src1_v, dst1_v, isem1)
                run_block(xw_hbm, src1_v, dst1_v, with_deg)

            if with_deg:
                @pl.loop(0, _NBLK * _BC)
                def _(i):
                    pltpu.make_async_copy(ones_v, dacc.at[dst1_v.at[0]],
                                          dsem).wait()

        pltpu.sync_copy(ones_hbm.at[pl.ds(0, _C)], ones_v)
        pltpu.sync_copy(xw1_hbm.at[rows], acc.at[rows])
        pltpu.sync_copy(ones_hbm.at[rows], dacc.at[rows])
        plsc.subcore_barrier()
        edge_pass(xw1_hbm, with_deg=True)
        plsc.subcore_barrier()
        pltpu.sync_copy(acc.at[rows], out_hbm.at[0, rel, rows])
        pltpu.sync_copy(dacc.at[rows], deg_hbm.at[rel, rows])
        pltpu.sync_copy(xw2_hbm.at[rows], acc.at[rows])
        plsc.subcore_barrier()
        edge_pass(xw2_hbm, with_deg=False)
        plsc.subcore_barrier()
        pltpu.sync_copy(acc.at[rows], out_hbm.at[1, rel, rows])

    return k(xw1, xw2, adjs_r, ones16)


def _att(sums, degP, Wsem, q2):
    def body(s_ref, d_ref, wsem_ref, q_ref, h_ref, sm_ref):
        dinv = 1.0 / d_ref[0]
        h = jnp.maximum(s_ref[0, 0] * dinv, 0.0)
        h_ref[0, 0] = h
        t = jnp.tanh(jnp.dot(h, wsem_ref[...], preferred_element_type=jnp.float32))
        att = jnp.dot(t, q_ref[...], preferred_element_type=jnp.float32)
        sm_ref[0, 0] = jnp.mean(att, axis=0, keepdims=True)

    return pl.pallas_call(
        body,
        grid=(2, _P),
        in_specs=[
            pl.BlockSpec((1, 1, _N, _F), lambda si, p: (si, p, 0, 0)),
            pl.BlockSpec((1, _N, 1), lambda si, p: (p, 0, 0)),
            pl.BlockSpec((_F, _SH), lambda si, p: (0, 0)),
            pl.BlockSpec((_SH, 1), lambda si, p: (0, 0)),
        ],
        out_specs=[
            pl.BlockSpec((1, 1, _N, _F), lambda si, p: (si, p, 0, 0)),
            pl.BlockSpec((1, 1, 1, 1), lambda si, p: (si, p, 0, 0)),
        ],
        out_shape=(jax.ShapeDtypeStruct((2, _P, _N, _F), jnp.float32),
                   jax.ShapeDtypeStruct((2, _P, 1, 1), jnp.float32)),
    )(sums, degP, Wsem, q2)


def _combine(hst, sm, init0, ct):
    def body(h_ref, sm_ref, mu0_ref, ct_ref,
             h1_ref, h2_ref, c_ref, mu_ref, r_ref, dist_ref):
        h1 = None
        for si in range(2):
            s0 = sm_ref[si, 0]
            s1 = sm_ref[si, 1]
            m = jnp.maximum(s0, s1)
            e0 = jnp.exp(s0 - m)
            e1 = jnp.exp(s1 - m)
            h = (e0 * h_ref[si, 0] + e1 * h_ref[si, 1]) / (e0 + e1)
            if si == 0:
                h1 = h
                h1_ref[...] = h
                c_ref[...] = jax.nn.sigmoid(jnp.mean(h, axis=0, keepdims=True))
            else:
                h2_ref[...] = h
        ct = ct_ref[...]

        nrm = lax.rsqrt(jnp.sum(h1 * h1, axis=1, keepdims=True))
        data = h1 * nrm
        ones_col = jnp.ones((_N, 1), jnp.float32)
        mu = mu0_ref[...]
        for _ in range(2):
            dist = lax.dot_general(data, mu, (((1,), (1,)), ((), ())),
                                   preferred_element_type=jnp.float32)
            z = ct * dist
            ex = jnp.exp(z - jnp.max(z, axis=1, keepdims=True))
            r = ex / jnp.sum(ex, axis=1, keepdims=True)
            cm = lax.dot_general(r, data, (((0,), (0,)), ((), ())),
                                 preferred_element_type=jnp.float32)
            cr = lax.dot_general(r, ones_col, (((0,), (0,)), ((), ())),
                                 preferred_element_type=jnp.float32)
            mu = cm / cr
        dist = lax.dot_general(data, mu, (((1,), (1,)), ((), ())),
                               preferred_element_type=jnp.float32)
        z = ct * dist
        ex = jnp.exp(z - jnp.max(z, axis=1, keepdims=True))
        r = ex / jnp.sum(ex, axis=1, keepdims=True)
        mu_ref[...] = mu
        r_ref[...] = r
        dist_ref[...] = dist

    return pl.pallas_call(
        body,
        out_shape=(jax.ShapeDtypeStruct((_N, _F), jnp.float32),
                   jax.ShapeDtypeStruct((_N, _F), jnp.float32),
                   jax.ShapeDtypeStruct((1, _F), jnp.float32),
                   jax.ShapeDtypeStruct((_KC, _F), jnp.float32),
                   jax.ShapeDtypeStruct((_N, _KC), jnp.float32),
                   jax.ShapeDtypeStruct((_N, _KC), jnp.float32)),
        compiler_params=pltpu.CompilerParams(vmem_limit_bytes=63 << 20),
    )(hst, sm, init0, ct)


def kernel(seq1, seq2, adjs, sparse, msk, samp_bias1, samp_bias2, K,
           cluster_temp, W, Wsem, q):
    x1 = seq1[0]
    x2 = seq2[0]
    xw1, xw2 = _pre_matmul(x1, x2, W)
    adjs_r = adjs.reshape(_P, 2, _NT, _NCH, _C)
    ones16 = jnp.ones((_N, 16), jnp.float32)
    sums, degs = _sc_msgpass(xw1, xw2, adjs_r, ones16)
    degP = degs[:, :, 0:1]
    ct = jnp.asarray(cluster_temp, jnp.float32).reshape(1, 1)
    init0 = jax.random.uniform(jax.random.key(42), (_KC, _F), dtype=jnp.float32)
    hst, sm = _att(sums, degP, Wsem, q.reshape(_SH, 1))
    h1, h2, c, mu, r, dist = _combine(hst, sm, init0, ct)
    return (h1[None], h2[None], c, mu, r, dist)
